# baseline (speedup 1.0000x reference)
import jax
import jax.numpy as jnp
from jax.experimental import pallas as pl
from jax.experimental.pallas import tpu as pltpu

N = 500000
C = 100
NBINS = 15
RB = 5000
NBLK = N // RB
PAD_ROWS = 4096
PADN = PAD_ROWS * 128
NSLOT = 32
SEARCH_ITERS = 30


def _stage1_kernel(x_ref, lab_ref, conf_ref, acc_ref):
    x = x_ref[...]
    lab = lab_ref[0, 0, :]
    m = jnp.max(x, axis=1)
    cols = jax.lax.broadcasted_iota(jnp.int32, (RB, C), 1)
    labc = lab[:, None]
    sel = jnp.where(cols == labc, 1.0,
                    jnp.where(cols < labc, 200.0, 0.0))
    w = jnp.where(x == m[:, None], sel, 0.0)
    s = jax.lax.dot_general(w, jnp.ones((C, 1), jnp.float32),
                            (((1,), (0,)), ((), ())),
                            preferred_element_type=jnp.float32)[:, 0]
    conf_ref[0, 0, :] = m
    acc_ref[0, 0, :] = jnp.where(s == 1.0, 1.0, 0.0)


def _stage2_kernel(ranks_ref, fracs_ref, conf_ref, acc_ref, out_ref,
                   lo_ref, hi_ref, edge_ref):
    conf = conf_ref[...]
    keys = jax.lax.bitcast_convert_type(conf, jnp.int32)

    def init(t, c):
        lo_ref[t] = jnp.int32(0)
        hi_ref[t] = jnp.int32(0x3F800000)
        return c
    jax.lax.fori_loop(0, NSLOT, init, 0)

    def step(s, c):
        def per_t(t, c2):
            lo = lo_ref[t]
            hi = hi_ref[t]
            mid = lo + (hi - lo) // 2
            cnt = jnp.sum((keys <= mid).astype(jnp.int32))
            ge = cnt >= ranks_ref[t] + 1
            lo_ref[t] = jnp.where(ge, lo, mid + 1)
            hi_ref[t] = jnp.where(ge, mid, hi)
            return c2
        return jax.lax.fori_loop(0, NSLOT, per_t, c)
    jax.lax.fori_loop(0, SEARCH_ITERS, step, 0)

    def mke(k, c):
        a = jax.lax.bitcast_convert_type(lo_ref[2 * k], jnp.float32)
        b = jax.lax.bitcast_convert_type(lo_ref[2 * k + 1], jnp.float32)
        edge_ref[k] = a + fracs_ref[k] * (b - a)
        return c
    jax.lax.fori_loop(0, NBINS + 1, mke, 0)

    acc = acc_ref[...]

    def binloop(b, tot):
        lo = edge_ref[b]
        up = edge_ref[b + 1]
        msk = (conf > lo) & (conf <= up)
        cnt = jnp.sum(jnp.where(msk, 1.0, 0.0))
        sc = jnp.sum(jnp.where(msk, conf, 0.0))
        sa = jnp.sum(jnp.where(msk, acc, 0.0))
        safe = jnp.maximum(cnt, 1.0)
        contrib = jnp.where(cnt > 0.0,
                            jnp.abs(sc / safe - sa / safe) * (cnt / N), 0.0)
        return tot + contrib
    ece = jax.lax.fori_loop(0, NBINS, binloop, jnp.float32(0.0))
    out_ref[0] = ece


def _stage1(softmax_in, labels_i32):
    lab3 = labels_i32.reshape(NBLK, 1, RB)
    conf3, acc3 = pl.pallas_call(
        _stage1_kernel,
        grid=(NBLK,),
        in_specs=[
            pl.BlockSpec((RB, C), lambda i: (i, 0)),
            pl.BlockSpec((1, 1, RB), lambda i: (i, 0, 0)),
        ],
        out_specs=[
            pl.BlockSpec((1, 1, RB), lambda i: (i, 0, 0)),
            pl.BlockSpec((1, 1, RB), lambda i: (i, 0, 0)),
        ],
        out_shape=[
            jax.ShapeDtypeStruct((NBLK, 1, RB), jnp.float32),
            jax.ShapeDtypeStruct((NBLK, 1, RB), jnp.float32),
        ],
    )(softmax_in, lab3)
    return conf3.reshape(N), acc3.reshape(N)


def _stage2(conf, acc, ranks, fracs):
    conf_p = jnp.pad(conf, (0, PADN - N),
                     constant_values=jnp.inf).reshape(PAD_ROWS, 128)
    acc_p = jnp.pad(acc, (0, PADN - N)).reshape(PAD_ROWS, 128)
    out = pl.pallas_call(
        _stage2_kernel,
        in_specs=[
            pl.BlockSpec(memory_space=pltpu.SMEM),
            pl.BlockSpec(memory_space=pltpu.SMEM),
            pl.BlockSpec((PAD_ROWS, 128), lambda: (0, 0)),
            pl.BlockSpec((PAD_ROWS, 128), lambda: (0, 0)),
        ],
        out_specs=pl.BlockSpec(memory_space=pltpu.SMEM),
        out_shape=jax.ShapeDtypeStruct((1,), jnp.float32),
        scratch_shapes=[
            pltpu.SMEM((NSLOT,), jnp.int32),
            pltpu.SMEM((NSLOT,), jnp.int32),
            pltpu.SMEM((NBINS + 1,), jnp.float32),
        ],
    )(ranks, fracs, conf_p, acc_p)
    return out


def kernel(softmax_in, labels):
    labels_i32 = labels.astype(jnp.int32)
    conf, acc = _stage1(softmax_in, labels_i32)

    q = jnp.linspace(0.0, float(N), NBINS + 1)
    iq = jnp.floor(q).astype(jnp.int32)
    frac = q - iq.astype(jnp.float32)
    oob = q >= jnp.float32(N - 1)
    frac = jnp.where(oob, 0.0, frac).astype(jnp.float32)
    lo_rank = jnp.where(oob, N - 1, jnp.clip(iq, 0, N - 1))
    hi_rank = jnp.minimum(lo_rank + 1, N - 1)
    ranks = jnp.stack([lo_rank, hi_rank], axis=1).reshape(NSLOT).astype(jnp.int32)

    return _stage2(conf, acc, ranks, fracs=frac)


# speedup vs baseline: 1.0420x; 1.0420x over previous
import functools

import jax
import jax.numpy as jnp
from jax import lax
from jax.experimental import pallas as pl
from jax.experimental.pallas import tpu as pltpu
from jax.experimental.pallas import tpu_sc as plsc

N = 500000
C = 100
NBINS = 15
RB = 5000
NBLK = N // RB
PAD_ROWS = 4096
PADN = PAD_ROWS * 128
NSLOT = 32
LB = 20
NB = 1 << (30 - LB)
SEARCH_ITERS = LB

NCORE = 2
NTILE = 16
SC_CHUNK = 15744
SC_ROWS = SC_CHUNK // 128
SC_NPAD = NCORE * NTILE * SC_CHUNK
PAD_KEY = 0x3FFFFFFF


def _sc_hist_body(keys_hbm, out_hbm, kbuf, idxbuf, ones_ref, zbuf, shared):
    cid = lax.axis_index("c")
    sid = lax.axis_index("s")
    wid = cid * NTILE + sid
    pltpu.sync_copy(keys_hbm.at[pl.ds(wid * SC_CHUNK, SC_CHUNK)], kbuf)

    zwords = NB // NTILE
    zv = jnp.zeros((16,), jnp.int32)

    def zfill(i, c):
        zbuf[pl.ds(i * 16, 16)] = zv
        return c
    lax.fori_loop(0, zwords // 16, zfill, 0)
    pltpu.sync_copy(zbuf, shared.at[pl.ds(sid * zwords, zwords)])

    def ofill(i, c):
        ones_ref[pl.ds(i * 16, 16)] = jnp.full((16,), 1, jnp.int32)
        return c
    lax.fori_loop(0, 8, ofill, 0)
    plsc.subcore_barrier()

    def block(r, c):
        def sub(i, c2):
            j = r * 8 + i
            k = kbuf[pl.ds(j * 16, 16)]
            idxbuf[pl.ds(i * 16, 16)] = k >> LB
            return c2
        lax.fori_loop(0, 8, sub, 0)
        pltpu.sync_copy(ones_ref, shared.at[idxbuf], add=True)
        return c
    lax.fori_loop(0, SC_ROWS, block, 0)

    plsc.subcore_barrier()

    @pl.when(sid == 0)
    def _():
        pltpu.sync_copy(shared, out_hbm.at[cid])


def _sc_hist(keys_p):
    mesh = plsc.VectorSubcoreMesh(core_axis_name="c", subcore_axis_name="s")
    f = functools.partial(
        pl.kernel,
        mesh=mesh,
        out_type=jax.ShapeDtypeStruct((NCORE, NB), jnp.int32),
        scratch_types=[
            pltpu.VMEM((SC_CHUNK,), jnp.int32),
            pltpu.VMEM((128,), jnp.int32),
            pltpu.VMEM((128,), jnp.int32),
            pltpu.VMEM((NB // NTILE,), jnp.int32),
            pltpu.VMEM_SHARED((NB,), jnp.int32),
        ],
    )(_sc_hist_body)
    return f(keys_p)


def _locate_kernel(ranks_ref, h_ref, b1_ref):
    h = h_ref[...]
    tot = (h[0:1, :] + h[1:2, :]).astype(jnp.float32)
    rows_i = lax.broadcasted_iota(jnp.int32, (NB, NB), 0)
    cols_i = lax.broadcasted_iota(jnp.int32, (NB, NB), 1)
    tri = jnp.where(rows_i <= cols_i, 1.0, 0.0)
    cum = lax.dot_general(tot, tri, (((1,), (0,)), ((), ())),
                          precision=lax.Precision.HIGHEST,
                          preferred_element_type=jnp.float32)

    def per_t(t, c):
        r = ranks_ref[t].astype(jnp.float32)
        b1_ref[t] = jnp.sum(jnp.where(cum <= r, 1, 0))
        return c
    lax.fori_loop(0, NSLOT, per_t, 0)


def _locate(h1, ranks):
    return pl.pallas_call(
        _locate_kernel,
        in_specs=[pl.BlockSpec(memory_space=pltpu.SMEM),
                  pl.BlockSpec((NCORE, NB), lambda: (0, 0))],
        out_specs=pl.BlockSpec(memory_space=pltpu.SMEM),
        out_shape=jax.ShapeDtypeStruct((NSLOT,), jnp.int32),
    )(ranks, h1)


def _stage1_kernel(x_ref, lab_ref, conf_ref, acc_ref):
    x = x_ref[...]
    lab = lab_ref[0, 0, :]
    m = jnp.max(x, axis=1)
    cols = jax.lax.broadcasted_iota(jnp.int32, (RB, C), 1)
    labc = lab[:, None]
    sel = jnp.where(cols == labc, 1.0,
                    jnp.where(cols < labc, 200.0, 0.0))
    w = jnp.where(x == m[:, None], sel, 0.0)
    s = jax.lax.dot_general(w, jnp.ones((C, 1), jnp.float32),
                            (((1,), (0,)), ((), ())),
                            preferred_element_type=jnp.float32)[:, 0]
    conf_ref[0, 0, :] = m
    acc_ref[0, 0, :] = jnp.where(s == 1.0, 1.0, 0.0)


def _stage2_kernel(ranks_ref, fracs_ref, b1_ref, conf_ref, acc_ref, out_ref,
                   lo_ref, hi_ref, edge_ref):
    conf = conf_ref[...]
    keys = jax.lax.bitcast_convert_type(conf, jnp.int32)

    def init(t, c):
        lo_ref[t] = b1_ref[t] << LB
        hi_ref[t] = ((b1_ref[t] + 1) << LB) - 1
        return c
    jax.lax.fori_loop(0, NSLOT, init, 0)

    def step(s, c):
        def per_t(t, c2):
            lo = lo_ref[t]
            hi = hi_ref[t]
            mid = lo + (hi - lo) // 2
            cnt = jnp.sum((keys <= mid).astype(jnp.int32))
            ge = cnt >= ranks_ref[t] + 1
            lo_ref[t] = jnp.where(ge, lo, mid + 1)
            hi_ref[t] = jnp.where(ge, mid, hi)
            return c2
        return jax.lax.fori_loop(0, NSLOT, per_t, c)
    jax.lax.fori_loop(0, SEARCH_ITERS, step, 0)

    def mke(k, c):
        a = jax.lax.bitcast_convert_type(lo_ref[2 * k], jnp.float32)
        b = jax.lax.bitcast_convert_type(lo_ref[2 * k + 1], jnp.float32)
        edge_ref[k] = a + fracs_ref[k] * (b - a)
        return c
    jax.lax.fori_loop(0, NBINS + 1, mke, 0)

    acc = acc_ref[...]

    def binloop(b, tot):
        lo = edge_ref[b]
        up = edge_ref[b + 1]
        msk = (conf > lo) & (conf <= up)
        cnt = jnp.sum(jnp.where(msk, 1.0, 0.0))
        sc = jnp.sum(jnp.where(msk, conf, 0.0))
        sa = jnp.sum(jnp.where(msk, acc, 0.0))
        safe = jnp.maximum(cnt, 1.0)
        contrib = jnp.where(cnt > 0.0,
                            jnp.abs(sc / safe - sa / safe) * (cnt / N), 0.0)
        return tot + contrib
    ece = jax.lax.fori_loop(0, NBINS, binloop, jnp.float32(0.0))
    out_ref[0] = ece


def _stage1(softmax_in, labels_i32):
    lab3 = labels_i32.reshape(NBLK, 1, RB)
    conf3, acc3 = pl.pallas_call(
        _stage1_kernel,
        grid=(NBLK,),
        in_specs=[
            pl.BlockSpec((RB, C), lambda i: (i, 0)),
            pl.BlockSpec((1, 1, RB), lambda i: (i, 0, 0)),
        ],
        out_specs=[
            pl.BlockSpec((1, 1, RB), lambda i: (i, 0, 0)),
            pl.BlockSpec((1, 1, RB), lambda i: (i, 0, 0)),
        ],
        out_shape=[
            jax.ShapeDtypeStruct((NBLK, 1, RB), jnp.float32),
            jax.ShapeDtypeStruct((NBLK, 1, RB), jnp.float32),
        ],
    )(softmax_in, lab3)
    return conf3.reshape(N), acc3.reshape(N)


def _stage2(conf, acc, ranks, fracs, b1s):
    conf_p = jnp.pad(conf, (0, PADN - N),
                     constant_values=jnp.inf).reshape(PAD_ROWS, 128)
    acc_p = jnp.pad(acc, (0, PADN - N)).reshape(PAD_ROWS, 128)
    out = pl.pallas_call(
        _stage2_kernel,
        in_specs=[
            pl.BlockSpec(memory_space=pltpu.SMEM),
            pl.BlockSpec(memory_space=pltpu.SMEM),
            pl.BlockSpec(memory_space=pltpu.SMEM),
            pl.BlockSpec((PAD_ROWS, 128), lambda: (0, 0)),
            pl.BlockSpec((PAD_ROWS, 128), lambda: (0, 0)),
        ],
        out_specs=pl.BlockSpec(memory_space=pltpu.SMEM),
        out_shape=jax.ShapeDtypeStruct((1,), jnp.float32),
        scratch_shapes=[
            pltpu.SMEM((NSLOT,), jnp.int32),
            pltpu.SMEM((NSLOT,), jnp.int32),
            pltpu.SMEM((NBINS + 1,), jnp.float32),
        ],
    )(ranks, fracs, b1s, conf_p, acc_p)
    return out


def kernel(softmax_in, labels):
    labels_i32 = labels.astype(jnp.int32)
    conf, acc = _stage1(softmax_in, labels_i32)

    q = jnp.linspace(0.0, float(N), NBINS + 1)
    iq = jnp.floor(q).astype(jnp.int32)
    frac = q - iq.astype(jnp.float32)
    oob = q >= jnp.float32(N - 1)
    frac = jnp.where(oob, 0.0, frac).astype(jnp.float32)
    lo_rank = jnp.where(oob, N - 1, jnp.clip(iq, 0, N - 1))
    hi_rank = jnp.minimum(lo_rank + 1, N - 1)
    ranks = jnp.stack([lo_rank, hi_rank], axis=1).reshape(NSLOT).astype(jnp.int32)

    keys = jax.lax.bitcast_convert_type(conf, jnp.int32)
    keys_p = jnp.pad(keys, (0, SC_NPAD - N), constant_values=PAD_KEY)
    h1 = _sc_hist(keys_p)
    b1s = _locate(h1, ranks)

    return _stage2(conf, acc, ranks, frac, b1s)
